# Initial kernel scaffold; baseline (speedup 1.0000x reference)
import jax
import jax.numpy as jnp
from jax.experimental import pallas as pl


def kernel(x, edge_index, W1, b1, W2, b2):
    raise NotImplementedError("write your pallas kernel here")


# speedup vs baseline: 7.8470x; 7.8470x over previous
import functools

import jax
import jax.numpy as jnp
from jax import lax
from jax.experimental import pallas as pl
from jax.experimental.pallas import tpu as pltpu
from jax.experimental.pallas import tpu_sc as plsc

NC = 2
NS = 16
LANES = 16
CH = 128
G = 16


def _fill_loop(ref, nrows, ncols, val):
    k = ncols // LANES
    v = jnp.full((LANES,), val, jnp.float32)

    def body(i, _):
        ref[i // k, pl.ds((i % k) * LANES, LANES)] = v
        return 0

    lax.fori_loop(0, nrows * k, body, 0)


def _make_deg_kernel(npad, nchunk):
    mesh = plsc.VectorSubcoreMesh(
        core_axis_name="c", subcore_axis_name="s", num_cores=NC, num_subcores=NS
    )
    rpt = npad // NS

    @functools.partial(
        pl.kernel,
        out_type=[jax.ShapeDtypeStruct((npad, G), jnp.float32)] * NC,
        mesh=mesh,
        scratch_types=[
            pltpu.VMEM((nchunk, CH), jnp.int32),
            pltpu.VMEM((CH, G), jnp.float32),
            pltpu.VMEM_SHARED((npad, G), jnp.float32),
        ],
    )
    def deg_kernel(dst_hbm, out0, out1, dst_v, buf_v, acc_sh):
        c = lax.axis_index("c")
        s = lax.axis_index("s")
        w = c * NS + s

        _fill_loop(buf_v, CH, G, 0.0)
        for kk in range(rpt // CH):
            pltpu.sync_copy(buf_v, acc_sh.at[pl.ds(s * rpt + kk * CH, CH)])
        plsc.subcore_barrier()

        _fill_loop(buf_v, CH, G, 1.0)
        pltpu.sync_copy(dst_hbm.at[w], dst_v)

        def body(j, _):
            pltpu.sync_copy(buf_v, acc_sh.at[dst_v.at[j]], add=True)
            return 0

        lax.fori_loop(0, nchunk, body, 0)
        plsc.subcore_barrier()

        @pl.when(c == 0)
        def _():
            pltpu.sync_copy(acc_sh.at[pl.ds(s * rpt, rpt)], out0.at[pl.ds(s * rpt, rpt)])

        @pl.when(c == 1)
        def _():
            pltpu.sync_copy(acc_sh.at[pl.ds(s * rpt, rpt)], out1.at[pl.ds(s * rpt, rpt)])

    return deg_kernel


def _make_edge_kernel(npad, d, nchunk):
    mesh = plsc.VectorSubcoreMesh(
        core_axis_name="c", subcore_axis_name="s", num_cores=NC, num_subcores=NS
    )
    rpt = npad // NS

    @functools.partial(
        pl.kernel,
        out_type=[jax.ShapeDtypeStruct((npad, d), jnp.float32)] * NC,
        mesh=mesh,
        scratch_types=[
            pltpu.VMEM((nchunk, CH), jnp.int32),
            pltpu.VMEM((nchunk, CH), jnp.int32),
            pltpu.VMEM((CH, d), jnp.float32),
            pltpu.VMEM((CH, d), jnp.float32),
            pltpu.VMEM_SHARED((npad, d), jnp.float32),
            pltpu.SemaphoreType.DMA,
        ],
    )
    def edge_kernel(
        hs_hbm, src_hbm, dst_hbm, out0, out1, src_v, dst_v, rows0_v, rows1_v, acc_sh, sem
    ):
        c = lax.axis_index("c")
        s = lax.axis_index("s")
        w = c * NS + s

        _fill_loop(rows0_v, CH, d, 0.0)
        for kk in range(rpt // CH):
            pltpu.sync_copy(rows0_v, acc_sh.at[pl.ds(s * rpt + kk * CH, CH)])
        plsc.subcore_barrier()

        pltpu.sync_copy(src_hbm.at[w], src_v)
        pltpu.sync_copy(dst_hbm.at[w], dst_v)

        def body(j, _):
            pltpu.sync_copy(hs_hbm.at[src_v.at[j]], rows0_v)
            pltpu.sync_copy(rows0_v, acc_sh.at[dst_v.at[j]], add=True)
            return 0

        lax.fori_loop(0, nchunk, body, 0)
        plsc.subcore_barrier()

        @pl.when(c == 0)
        def _():
            pltpu.sync_copy(acc_sh.at[pl.ds(s * rpt, rpt)], out0.at[pl.ds(s * rpt, rpt)])

        @pl.when(c == 1)
        def _():
            pltpu.sync_copy(acc_sh.at[pl.ds(s * rpt, rpt)], out1.at[pl.ds(s * rpt, rpt)])

    return edge_kernel


def _dinv_block(d0, d1):
    deg = d0[:, :1] + d1[:, :1] + 1.0
    return lax.rsqrt(deg)


def _tc_scale_matmul(x_p, w, d0, d1, bl=1024):
    npad, d = x_p.shape

    def body(x_ref, w_ref, d0_ref, d1_ref, o_ref):
        dinv = _dinv_block(d0_ref[...], d1_ref[...])
        o_ref[...] = (
            jnp.dot(x_ref[...], w_ref[...], preferred_element_type=jnp.float32) * dinv
        )

    return pl.pallas_call(
        body,
        grid=(npad // bl,),
        in_specs=[
            pl.BlockSpec((bl, d), lambda i: (i, 0)),
            pl.BlockSpec((d, d), lambda i: (0, 0)),
            pl.BlockSpec((bl, G), lambda i: (i, 0)),
            pl.BlockSpec((bl, G), lambda i: (i, 0)),
        ],
        out_specs=pl.BlockSpec((bl, d), lambda i: (i, 0)),
        out_shape=jax.ShapeDtypeStruct((npad, d), jnp.float32),
    )(x_p, w, d0, d1)


def _tc_combine_matmul(a0, a1, hs, d0, d1, w, b, bl=1024):
    npad, d = hs.shape

    def body(a0_ref, a1_ref, hs_ref, d0_ref, d1_ref, w_ref, b_ref, o_ref):
        dinv = _dinv_block(d0_ref[...], d1_ref[...])
        t = jnp.maximum(
            dinv * (a0_ref[...] + a1_ref[...] + hs_ref[...]) + b_ref[...], 0.0
        )
        o_ref[...] = (
            jnp.dot(t, w_ref[...], preferred_element_type=jnp.float32) * dinv
        )

    return pl.pallas_call(
        body,
        grid=(npad // bl,),
        in_specs=[
            pl.BlockSpec((bl, d), lambda i: (i, 0)),
            pl.BlockSpec((bl, d), lambda i: (i, 0)),
            pl.BlockSpec((bl, d), lambda i: (i, 0)),
            pl.BlockSpec((bl, G), lambda i: (i, 0)),
            pl.BlockSpec((bl, G), lambda i: (i, 0)),
            pl.BlockSpec((d, d), lambda i: (0, 0)),
            pl.BlockSpec((1, d), lambda i: (0, 0)),
        ],
        out_specs=pl.BlockSpec((bl, d), lambda i: (i, 0)),
        out_shape=jax.ShapeDtypeStruct((npad, d), jnp.float32),
    )(a0, a1, hs, d0, d1, w, b)


def _tc_combine(a0, a1, hs, d0, d1, b, bl=1024):
    npad, d = hs.shape

    def body(a0_ref, a1_ref, hs_ref, d0_ref, d1_ref, b_ref, o_ref):
        dinv = _dinv_block(d0_ref[...], d1_ref[...])
        o_ref[...] = jnp.maximum(
            dinv * (a0_ref[...] + a1_ref[...] + hs_ref[...]) + b_ref[...], 0.0
        )

    return pl.pallas_call(
        body,
        grid=(npad // bl,),
        in_specs=[
            pl.BlockSpec((bl, d), lambda i: (i, 0)),
            pl.BlockSpec((bl, d), lambda i: (i, 0)),
            pl.BlockSpec((bl, d), lambda i: (i, 0)),
            pl.BlockSpec((bl, G), lambda i: (i, 0)),
            pl.BlockSpec((bl, G), lambda i: (i, 0)),
            pl.BlockSpec((1, d), lambda i: (0, 0)),
        ],
        out_specs=pl.BlockSpec((bl, d), lambda i: (i, 0)),
        out_shape=jax.ShapeDtypeStruct((npad, d), jnp.float32),
    )(a0, a1, hs, d0, d1, b)


def kernel(x, edge_index, W1, b1, W2, b2):
    n, d = x.shape
    e = edge_index.shape[1]
    nw = NC * NS

    npad = -(-(n + 1) // 2048) * 2048
    per_w = -(-e // (nw * 2 * CH)) * 2 * CH
    nchunk = per_w // CH
    e_pad = per_w * nw

    src = edge_index[0]
    dst = edge_index[1]
    pad = e_pad - e
    if pad:
        src = jnp.concatenate([src, jnp.zeros((pad,), jnp.int32)])
        dst = jnp.concatenate([dst, jnp.full((pad,), n, jnp.int32)])
    src_m = src.reshape(nw, nchunk, CH)
    dst_m = dst.reshape(nw, nchunk, CH)
    x_p = jnp.pad(x, ((0, npad - n), (0, 0)))
    b1r = b1.reshape(1, d)
    b2r = b2.reshape(1, d)

    d0, d1 = _make_deg_kernel(npad, nchunk)(dst_m)
    edge_k = _make_edge_kernel(npad, d, nchunk)

    hs1 = _tc_scale_matmul(x_p, W1, d0, d1)
    a0, a1 = edge_k(hs1, src_m, dst_m)
    hs2 = _tc_combine_matmul(a0, a1, hs1, d0, d1, W2, b1r)
    c0, c1 = edge_k(hs2, src_m, dst_m)
    out = _tc_combine(c0, c1, hs2, d0, d1, b2r)
    return out[:n]
